# baseline (speedup 1.0000x reference)
import jax
import jax.numpy as jnp
from jax.experimental import pallas as pl

_SLAB = 8


def _corner_kernel(x_ref, o_ref):
    cb, h, w = x_ref.shape
    wa = 128
    wb = w - wa
    neg = -jnp.inf
    carry = jnp.full((cb, 1, w), neg, x_ref.dtype)
    for g in range(h // _SLAB):
        r0 = g * _SLAB
        v = x_ref[:, r0:r0 + _SLAB, :]
        for k in (1, 2, 4):
            pad = jnp.full((cb, k, w), neg, v.dtype)
            v = jnp.maximum(v, jnp.concatenate([pad, v[:, :-k, :]], axis=1))
        v = jnp.maximum(v, carry)
        carry = v[:, _SLAB - 1:_SLAB, :]
        a = v[:, :, :wa]
        b = v[:, :, wa:]
        for k in (1, 2, 4, 8, 16, 32, 64):
            pad_a = jnp.full((cb, _SLAB, k), neg, v.dtype)
            a = jnp.maximum(a, jnp.concatenate([pad_a, a[:, :, :-k]], axis=2))
            if k < wb:
                b = jnp.maximum(b, jnp.concatenate([pad_a, b[:, :, :-k]], axis=2))
        b = jnp.maximum(b, a[:, :, wa - 1:wa])
        o_ref[:, r0:r0 + _SLAB, :wa] = a + a
        o_ref[:, r0:r0 + _SLAB, wa:] = b + b


def kernel(x):
    b, c, h, w = x.shape
    xf = x.reshape(b * c, h, w)
    cb = 8
    out = pl.pallas_call(
        _corner_kernel,
        grid=((b * c) // cb,),
        in_specs=[pl.BlockSpec((cb, h, w), lambda i: (i, 0, 0))],
        out_specs=pl.BlockSpec((cb, h, w), lambda i: (i, 0, 0)),
        out_shape=jax.ShapeDtypeStruct((b * c, h, w), x.dtype),
    )(xf)
    return out.reshape(b, c, h, w)


# speedup vs baseline: 4.9438x; 4.9438x over previous
import functools

import jax
import jax.numpy as jnp
from jax import lax
from jax.experimental import pallas as pl
from jax.experimental.pallas import tpu as pltpu
from jax.experimental.pallas import tpu_sc as plsc

_H = 224
_W = 224
_L = 16
_NCOL = _W // _L
_NWORK = 32


def _sc_corner(x_hbm, o_hbm, buf, in_sem, out_sem):
    wid = lax.axis_index("s") * 2 + lax.axis_index("c")
    n_img = x_hbm.shape[0] // _NWORK
    neg16 = jnp.full((_L,), -jnp.inf, jnp.float32)

    def row_body(h, hcs):
        cw = neg16
        out_hcs = []
        for j in range(_NCOL):
            v = buf[h, pl.ds(j * _L, _L)]
            hc = jnp.maximum(hcs[j], v)
            s = jnp.maximum(plsc.cummax(hc), cw)
            cw = jnp.maximum(cw, jnp.full((_L,), lax.reduce_max(hc, (0,))))
            buf[h, pl.ds(j * _L, _L)] = s + s
            out_hcs.append(hc)
        return tuple(out_hcs)

    for i in range(n_img):
        img = wid * n_img + i
        pltpu.sync_copy(x_hbm.at[img], buf)
        lax.fori_loop(0, _H, row_body, tuple([neg16] * _NCOL))
        pltpu.sync_copy(buf, o_hbm.at[img])


def kernel(x):
    b, c, h, w = x.shape
    xf = x.reshape(b * c, h, w)
    fn = functools.partial(
        pl.kernel,
        mesh=plsc.VectorSubcoreMesh(core_axis_name="c", subcore_axis_name="s"),
        out_type=jax.ShapeDtypeStruct((b * c, h, w), jnp.float32),
        scratch_types=[
            pltpu.VMEM((h, w), jnp.float32),
            pltpu.SemaphoreType.DMA,
            pltpu.SemaphoreType.DMA,
        ],
        compiler_params=pltpu.CompilerParams(needs_layout_passes=False),
    )(_sc_corner)
    return fn(xf).reshape(b, c, h, w)


# speedup vs baseline: 7.3403x; 1.4847x over previous
import functools

import jax
import jax.numpy as jnp
from jax import lax
from jax.experimental import pallas as pl
from jax.experimental.pallas import tpu as pltpu
from jax.experimental.pallas import tpu_sc as plsc

_H = 224
_W = 224
_L = 16
_NCOL = _W // _L
_NWORK = 32


def _sc_corner(x_hbm, o_hbm, buf, isem0, isem1, osem0, osem1):
    wid = lax.axis_index("s") * 2 + lax.axis_index("c")
    n_img = x_hbm.shape[0] // _NWORK
    neg16 = jnp.full((_L,), -jnp.inf, jnp.float32)
    isems = (isem0, isem1)
    osems = (osem0, osem1)

    def make_row_body(slot):
        def row_body(h, hcs):
            cw = neg16
            out_hcs = []
            for j in range(_NCOL):
                v = buf[slot, h, pl.ds(j * _L, _L)]
                hc = jnp.maximum(hcs[j], v)
                s = jnp.maximum(plsc.cummax(hc), cw)
                cw = jnp.maximum(cw, jnp.full((_L,),
                                              lax.reduce_max(hc, (0,))))
                buf[slot, h, pl.ds(j * _L, _L)] = s + s
                out_hcs.append(hc)
            return tuple(out_hcs)
        return row_body

    base = wid * n_img
    in_h = {}
    out_h = {}
    in_h[0] = pltpu.async_copy(x_hbm.at[base], buf.at[0], isems[0])
    for i in range(n_img):
        s = i % 2
        if i + 1 < n_img:
            if i >= 1:
                out_h[i - 1].wait()
            in_h[i + 1] = pltpu.async_copy(x_hbm.at[base + i + 1],
                                           buf.at[(i + 1) % 2],
                                           isems[(i + 1) % 2])
        in_h[i].wait()
        lax.fori_loop(0, _H, make_row_body(s), tuple([neg16] * _NCOL))
        out_h[i] = pltpu.async_copy(buf.at[s], o_hbm.at[base + i], osems[s])
    out_h[n_img - 2].wait()
    out_h[n_img - 1].wait()


def kernel(x):
    b, c, h, w = x.shape
    xf = x.reshape(b * c, h, w)
    fn = functools.partial(
        pl.kernel,
        mesh=plsc.VectorSubcoreMesh(core_axis_name="c", subcore_axis_name="s"),
        out_type=jax.ShapeDtypeStruct((b * c, h, w), jnp.float32),
        scratch_types=[
            pltpu.VMEM((2, h, w), jnp.float32),
            pltpu.SemaphoreType.DMA,
            pltpu.SemaphoreType.DMA,
            pltpu.SemaphoreType.DMA,
            pltpu.SemaphoreType.DMA,
        ],
        compiler_params=pltpu.CompilerParams(needs_layout_passes=False),
    )(_sc_corner)
    return fn(xf).reshape(b, c, h, w)
